# Initial kernel scaffold; baseline (speedup 1.0000x reference)
import functools

import jax
import jax.numpy as jnp
from jax import lax
from jax.experimental import pallas as pl
from jax.experimental.pallas import tpu as pltpu
from jax.experimental.pallas import tpu_sc as plsc

BATCH = 16384
N_FIELDS = 26
FIELD_DIM = 100000
TOTAL_ROWS = N_FIELDS * FIELD_DIM

NUM_CORES = 2
NUM_SUBCORES = 16
NUM_WORKERS = NUM_CORES * NUM_SUBCORES
ROWS_PER_W = BATCH // NUM_WORKERS
FLAT_PER_W = ROWS_PER_W * N_FIELDS
LANES = 16

_mesh = plsc.VectorSubcoreMesh(core_axis_name="c", subcore_axis_name="s")


@functools.partial(
    pl.kernel,
    mesh=_mesh,
    out_type=jax.ShapeDtypeStruct((BATCH,), jnp.float32),
    scratch_types=[
        pltpu.VMEM((FLAT_PER_W,), jnp.int32),
        pltpu.VMEM((FLAT_PER_W,), jnp.float32),
        pltpu.VMEM((ROWS_PER_W,), jnp.float32),
        pltpu.VMEM((LANES,), jnp.float32),
        pltpu.SemaphoreType.DMA,
    ],
)
def _lr_sc(data_hbm, w_hbm, bias_hbm, out_hbm, idx_v, vals_v, out_v, bias_v, sem):
    wid = lax.axis_index("s") * NUM_CORES + lax.axis_index("c")
    base = wid * FLAT_PER_W

    pltpu.sync_copy(data_hbm.at[pl.ds(base, FLAT_PER_W)], idx_v)
    pltpu.sync_copy(bias_hbm, bias_v)

    lane = lax.iota(jnp.int32, (LANES,))

    def add_offsets(i, carry):
        p0 = i * LANES
        pos = lane + p0
        idx_v[pl.ds(p0, LANES)] = idx_v[pl.ds(p0, LANES)] + (pos % N_FIELDS) * FIELD_DIM
        return carry

    lax.fori_loop(0, FLAT_PER_W // LANES, add_offsets, 0)

    pltpu.async_copy(w_hbm.at[idx_v], vals_v, sem).wait()

    bvec = bias_v[...]
    row16 = lane * N_FIELDS

    def reduce_block(blk, carry):
        b0 = blk * (LANES * N_FIELDS)
        acc = bvec
        for f in range(N_FIELDS):
            acc = acc + plsc.load_gather(vals_v, [row16 + (b0 + f)])
        out_v[pl.ds(blk * LANES, LANES)] = 1.0 / (1.0 + jnp.exp(-acc))
        return carry

    lax.fori_loop(0, ROWS_PER_W // LANES, reduce_block, 0)

    pltpu.sync_copy(out_v, out_hbm.at[pl.ds(wid * ROWS_PER_W, ROWS_PER_W)])


def kernel(data, W, bias):
    data_flat = data.reshape(-1).astype(jnp.int32)
    w_flat = W.reshape(-1)
    bias16 = jnp.broadcast_to(bias.astype(jnp.float32), (LANES,))
    return _lr_sc(data_flat, w_flat, bias16)


# speedup vs baseline: 1.1355x; 1.1355x over previous
import functools

import jax
import jax.numpy as jnp
from jax import lax
from jax.experimental import pallas as pl
from jax.experimental.pallas import tpu as pltpu
from jax.experimental.pallas import tpu_sc as plsc

BATCH = 16384
N_FIELDS = 26
FIELD_DIM = 100000
TOTAL_ROWS = N_FIELDS * FIELD_DIM

NUM_CORES = 2
NUM_SUBCORES = 16
NUM_WORKERS = NUM_CORES * NUM_SUBCORES
ROWS_PER_W = BATCH // NUM_WORKERS
FLAT_PER_W = ROWS_PER_W * N_FIELDS
LANES = 16

_mesh = plsc.VectorSubcoreMesh(core_axis_name="c", subcore_axis_name="s")


@functools.partial(
    pl.kernel,
    mesh=_mesh,
    out_type=jax.ShapeDtypeStruct((BATCH,), jnp.float32),
    compiler_params=pltpu.CompilerParams(needs_layout_passes=False),
    scratch_types=[
        pltpu.VMEM((FLAT_PER_W,), jnp.int32),
        pltpu.VMEM((FLAT_PER_W,), jnp.float32),
        pltpu.VMEM((ROWS_PER_W,), jnp.float32),
        pltpu.VMEM((LANES,), jnp.float32),
        pltpu.SemaphoreType.DMA,
    ],
)
def _lr_sc(data_hbm, w_hbm, bias_hbm, out_hbm, idx_v, vals_v, out_v, bias_v, sem):
    wid = lax.axis_index("s") * NUM_CORES + lax.axis_index("c")
    base = wid * FLAT_PER_W

    pltpu.sync_copy(data_hbm.at[pl.ds(base, FLAT_PER_W)], idx_v)
    pltpu.sync_copy(bias_hbm, bias_v)

    lane = lax.iota(jnp.int32, LANES)

    def add_offsets(i, carry):
        p0 = i * LANES
        pos = lane + p0
        idx_v[pl.ds(p0, LANES)] = idx_v[pl.ds(p0, LANES)] + (pos % N_FIELDS) * FIELD_DIM
        return carry

    lax.fori_loop(0, FLAT_PER_W // LANES, add_offsets, 0)

    pltpu.async_copy(w_hbm.at[idx_v], vals_v, sem).wait()

    bvec = bias_v[...]
    row16 = lane * N_FIELDS

    def reduce_block(blk, carry):
        b0 = blk * (LANES * N_FIELDS)
        acc = bvec
        for f in range(N_FIELDS):
            acc = acc + plsc.load_gather(vals_v, [row16 + (b0 + f)])
        out_v[pl.ds(blk * LANES, LANES)] = 1.0 / (1.0 + jnp.exp(-acc))
        return carry

    lax.fori_loop(0, ROWS_PER_W // LANES, reduce_block, 0)

    pltpu.sync_copy(out_v, out_hbm.at[pl.ds(wid * ROWS_PER_W, ROWS_PER_W)])


def kernel(data, W, bias):
    data_flat = data.reshape(-1).astype(jnp.int32)
    w_flat = W.reshape(-1)
    bias16 = jnp.broadcast_to(bias.astype(jnp.float32), (LANES,))
    return _lr_sc(data_flat, w_flat, bias16)


# speedup vs baseline: 1.1678x; 1.0284x over previous
import functools

import jax
import jax.numpy as jnp
from jax import lax
from jax.experimental import pallas as pl
from jax.experimental.pallas import tpu as pltpu
from jax.experimental.pallas import tpu_sc as plsc

BATCH = 16384
N_FIELDS = 26
FIELD_DIM = 100000
TOTAL_ROWS = N_FIELDS * FIELD_DIM

NUM_CORES = 2
NUM_SUBCORES = 16
NUM_WORKERS = NUM_CORES * NUM_SUBCORES
ROWS_PER_W = BATCH // NUM_WORKERS
FLAT_PER_W = ROWS_PER_W * N_FIELDS
LANES = 16

_mesh = plsc.VectorSubcoreMesh(core_axis_name="c", subcore_axis_name="s")


@functools.partial(
    pl.kernel,
    mesh=_mesh,
    out_type=jax.ShapeDtypeStruct((BATCH,), jnp.float32),
    compiler_params=pltpu.CompilerParams(
        needs_layout_passes=False, use_tc_tiling_on_sc=False
    ),
    scratch_types=[
        pltpu.VMEM((FLAT_PER_W,), jnp.int32),
        pltpu.VMEM((FLAT_PER_W,), jnp.float32),
        pltpu.VMEM((ROWS_PER_W,), jnp.float32),
        pltpu.VMEM((LANES,), jnp.float32),
        pltpu.SemaphoreType.DMA,
    ],
)
def _lr_sc(data_hbm, w_hbm, bias_hbm, out_hbm, idx_v, vals_v, out_v, bias_v, sem):
    wid = lax.axis_index("s") * NUM_CORES + lax.axis_index("c")
    base = wid * FLAT_PER_W

    pltpu.sync_copy(data_hbm.at[pl.ds(base, FLAT_PER_W)], idx_v)
    pltpu.sync_copy(bias_hbm, bias_v)

    lane = lax.iota(jnp.int32, LANES)

    pltpu.async_copy(w_hbm.at[idx_v], vals_v, sem).wait()

    bvec = bias_v[...]
    row16 = lane * N_FIELDS
    def reduce_block(blk, carry):
        b0 = blk * (LANES * N_FIELDS)
        acc = bvec
        for f in range(N_FIELDS):
            acc = acc + plsc.load_gather(vals_v, [row16 + (b0 + f)])
        out_v[pl.ds(blk * LANES, LANES)] = 1.0 / (1.0 + jnp.exp(-acc))
        return carry

    lax.fori_loop(0, ROWS_PER_W // LANES, reduce_block, 0)

    pltpu.sync_copy(out_v, out_hbm.at[pl.ds(wid * ROWS_PER_W, ROWS_PER_W)])


def kernel(data, W, bias):
    offsets = jnp.arange(N_FIELDS, dtype=data.dtype) * FIELD_DIM
    idx_flat = (data + offsets[None, :]).reshape(-1).astype(jnp.int32)
    one = jnp.where(bias[0] == bias[0], jnp.float32(1.0), jnp.float32(2.0))
    w_flat = W.reshape(-1) * one
    bias16 = jnp.broadcast_to(bias.astype(jnp.float32), (LANES,))
    return _lr_sc(idx_flat, w_flat, bias16)
